# baseline (device time: 50229 ns/iter reference)
import jax
import jax.numpy as jnp
from jax import lax
from jax.experimental import pallas as pl
from jax.experimental.pallas import tpu as pltpu

N_DEV = 4
CAP = 102
E_PER_DEV = 4
E_TOT = 16


def kernel(x, router_W, route_idx, expert_W):
    del router_W
    m_per, d = x.shape
    e_loc, _, h = expert_W.shape
    k_stk = e_loc * d

    def body(x_ref, route_ref, ew_ref, out_ref,
             comm_ref, my_cnt_ref, cnt_ref,
             cnt_send_sems, cnt_recv_sems, ring_send_sems, ring_recv_sems):
        my = lax.axis_index("i")
        right = lax.rem(my + 1, N_DEV)

        cnt_ref[...] = jnp.zeros_like(cnt_ref)

        barrier = pltpu.get_barrier_semaphore()
        for o in range(1, N_DEV):
            pl.semaphore_signal(
                barrier, inc=1,
                device_id=(lax.rem(my + o, N_DEV),),
                device_id_type=pl.DeviceIdType.MESH,
            )
        pl.semaphore_wait(barrier, N_DEV - 1)

        route = route_ref[:, 0]
        onehot = (
            route[:, None]
            == lax.broadcasted_iota(jnp.int32, (m_per, E_TOT), 1)
        ).astype(jnp.float32)
        my_cnt_ref[...] = jnp.sum(onehot, axis=0, keepdims=True)

        cnt_rdmas = []
        for o in range(1, N_DEV):
            peer = lax.rem(my + o, N_DEV)
            r = pltpu.make_async_remote_copy(
                src_ref=my_cnt_ref,
                dst_ref=cnt_ref.at[my],
                send_sem=cnt_send_sems.at[o - 1],
                recv_sem=cnt_recv_sems.at[my],
                device_id=(peer,),
                device_id_type=pl.DeviceIdType.MESH,
            )
            r.start()
            cnt_rdmas.append(r)

        comm_ref[0] = ew_ref[...].astype(jnp.bfloat16).reshape(k_stk, h)

        for o in range(1, N_DEV):
            peer = lax.rem(my + o, N_DEV)
            pltpu.make_async_remote_copy(
                src_ref=my_cnt_ref,
                dst_ref=cnt_ref.at[peer],
                send_sem=cnt_send_sems.at[0],
                recv_sem=cnt_recv_sems.at[peer],
                device_id=(peer,),
                device_id_type=pl.DeviceIdType.MESH,
            ).wait_recv()
        for r in cnt_rdmas:
            r.wait_send()

        row = lax.broadcasted_iota(jnp.int32, (m_per, m_per), 0)
        col = lax.broadcasted_iota(jnp.int32, (m_per, m_per), 1)
        tri = (col < row).astype(jnp.float32)
        prefix = jnp.dot(tri, onehot, preferred_element_type=jnp.float32)
        dev_w = (
            lax.broadcasted_iota(jnp.int32, (N_DEV, 1, E_TOT), 0) < my
        ).astype(jnp.float32)
        base = jnp.sum(cnt_ref[...] * dev_w, axis=0)
        rank = jnp.sum(onehot * (base + prefix), axis=1)
        keep = rank < CAP
        xb = x_ref[...].astype(jnp.bfloat16)

        def consume(slot, blk):
            src_dev = lax.rem(my - slot + N_DEV, N_DEV)
            parts = []
            for j in range(E_PER_DEV):
                e_id = src_dev * E_PER_DEV + j
                m = jnp.logical_and(route == e_id, keep).astype(jnp.bfloat16)
                parts.append(xb * m[:, None])
            xm = jnp.concatenate(parts, axis=1)
            return jnp.dot(xm, blk, preferred_element_type=jnp.float32)

        acc = jnp.zeros((m_per, h), jnp.float32)
        for hop in range(N_DEV - 1):
            rdma = pltpu.make_async_remote_copy(
                src_ref=comm_ref.at[hop],
                dst_ref=comm_ref.at[hop + 1],
                send_sem=ring_send_sems.at[hop],
                recv_sem=ring_recv_sems.at[hop],
                device_id=(right,),
                device_id_type=pl.DeviceIdType.MESH,
            )
            rdma.start()
            acc = acc + consume(hop, comm_ref[hop])
            rdma.wait()
        acc = acc + consume(N_DEV - 1, comm_ref[N_DEV - 1])
        out_ref[...] = acc

    return pl.pallas_call(
        body,
        out_shape=jax.ShapeDtypeStruct((m_per, h), jnp.float32),
        in_specs=[
            pl.BlockSpec(memory_space=pltpu.VMEM),
            pl.BlockSpec(memory_space=pltpu.VMEM),
            pl.BlockSpec(memory_space=pltpu.VMEM),
        ],
        out_specs=pl.BlockSpec(memory_space=pltpu.VMEM),
        scratch_shapes=[
            pltpu.VMEM((N_DEV, k_stk, h), jnp.bfloat16),
            pltpu.VMEM((1, E_TOT), jnp.float32),
            pltpu.VMEM((N_DEV, 1, E_TOT), jnp.float32),
            pltpu.SemaphoreType.DMA((N_DEV - 1,)),
            pltpu.SemaphoreType.DMA((N_DEV,)),
            pltpu.SemaphoreType.DMA((N_DEV - 1,)),
            pltpu.SemaphoreType.DMA((N_DEV - 1,)),
        ],
        compiler_params=pltpu.CompilerParams(collective_id=0),
    )(x, route_idx, expert_W)
